# Initial kernel scaffold; baseline (speedup 1.0000x reference)
import jax
import jax.numpy as jnp
from jax.experimental import pallas as pl


def kernel(x, edge_index, conv1_weight, conv1_bias, fc1_weight, fc1_bias, fc2_weight, fc2_bias, fc3_weight, fc3_bias, fc4_weight, fc4_bias):
    raise NotImplementedError("write your pallas kernel here")


# speedup vs baseline: 21.2784x; 21.2784x over previous
import jax
import jax.numpy as jnp
from jax import lax
from jax.experimental import pallas as pl
from jax.experimental.pallas import tpu as pltpu
from jax.experimental.pallas import tpu_sc as plsc

_N = 10000
_NP = 10240
_E = 320000
_DIN = 128
_D2 = 256
_DQ = 64
_NQ = 4
_NC = 2
_NS = 16
_NW = _NC * _NS
_RPT = _NP // _NS
_KA = 80
_CA = (_E // _NW) // _KA
_KC = 80
_CC = (_E // _NS) // _KC
_BN = 1000
_NB = _N // _BN

def _make_mesh():
    return plsc.VectorSubcoreMesh(core_axis_name="c", subcore_axis_name="s",
                                  num_cores=_NC, num_subcores=_NS)


def _deg_body(src_hbm, ones_hbm, zeros_hbm, deg_hbm, idx_v, ones_v, deg_sh):
    c = lax.axis_index("c")
    s = lax.axis_index("s")
    w = c * _NS + s
    pltpu.sync_copy(zeros_hbm.at[pl.ds(s * _RPT, _RPT), pl.ds(0, 16)],
                    deg_sh.at[pl.ds(s * _RPT, _RPT)])
    pltpu.sync_copy(ones_hbm, ones_v)
    pltpu.sync_copy(src_hbm.at[w], idx_v)
    plsc.subcore_barrier()

    def _chunk(j, carry):
        pltpu.sync_copy(ones_v, deg_sh.at[idx_v.at[j]], add=True)
        return carry

    lax.fori_loop(0, _CA, _chunk, None)
    plsc.subcore_barrier()
    pltpu.sync_copy(deg_sh.at[pl.ds(s * _RPT, _RPT)],
                    deg_hbm.at[c, pl.ds(s * _RPT, _RPT)])


def _deg_call(*args):
    return pl.kernel(
        _deg_body,
        out_type=jax.ShapeDtypeStruct((_NC, _NP, 16), jnp.float32),
        mesh=_make_mesh(),
        compiler_params=pltpu.CompilerParams(use_tc_tiling_on_sc=False),
        scratch_types=[
            pltpu.VMEM((_CA, _KA), jnp.int32),
            pltpu.VMEM((_KA, 16), jnp.float32),
            pltpu.VMEM_SHARED((_NP, 16), jnp.float32),
        ],
    )(*args)


def _mm_body(x_ref, w_ref, deg_ref, g4_ref):
    h = jnp.dot(x_ref[...], w_ref[...], preferred_element_type=jnp.float32)
    deg = deg_ref[0, :, 0:1] + deg_ref[1, :, 0:1] + 1.0
    g = lax.rsqrt(deg) * h
    for q in range(_NQ):
        g4_ref[q] = g[:, q * _DQ:(q + 1) * _DQ]


_mm_call = pl.pallas_call(
    _mm_body,
    grid=(_NB,),
    in_specs=[
        pl.BlockSpec((_BN, _DIN), lambda i: (i, 0)),
        pl.BlockSpec((_DIN, _D2), lambda i: (0, 0)),
        pl.BlockSpec((_NC, _BN, 16), lambda i: (0, i, 0)),
    ],
    out_specs=pl.BlockSpec((_NQ, _BN, _DQ), lambda i: (0, i, 0)),
    out_shape=jax.ShapeDtypeStruct((_NQ, _NP, _DQ), jnp.float32),
)


def _msg_body(g4_hbm, src_hbm, dst_hbm, zeros_hbm, acc_hbm,
              src_v, dst_v, rows0, rows1, acc_sh, sem0, sem1):
    c = lax.axis_index("c")
    s = lax.axis_index("s")
    pltpu.sync_copy(src_hbm.at[s], src_v)
    pltpu.sync_copy(dst_hbm.at[s], dst_v)

    for q in range(2):
        qg = c * 2 + q
        pltpu.sync_copy(zeros_hbm.at[pl.ds(s * _RPT, _RPT)],
                        acc_sh.at[pl.ds(s * _RPT, _RPT)])
        plsc.subcore_barrier()

        tbl = g4_hbm.at[qg]
        pltpu.async_copy(tbl.at[src_v.at[0]], rows0, sem0)
        pltpu.async_copy(tbl.at[src_v.at[1]], rows1, sem1)

        def _step(t, carry):
            j0 = 2 * t
            pltpu.make_async_copy(tbl.at[src_v.at[j0]], rows0, sem0).wait()
            pltpu.sync_copy(rows0, acc_sh.at[dst_v.at[j0]], add=True)

            @pl.when(j0 + 2 < _CC)
            def _():
                pltpu.async_copy(tbl.at[src_v.at[j0 + 2]], rows0, sem0)

            j1 = j0 + 1
            pltpu.make_async_copy(tbl.at[src_v.at[j1]], rows1, sem1).wait()
            pltpu.sync_copy(rows1, acc_sh.at[dst_v.at[j1]], add=True)

            @pl.when(j1 + 2 < _CC)
            def _():
                pltpu.async_copy(tbl.at[src_v.at[j1 + 2]], rows1, sem1)

            return carry

        lax.fori_loop(0, _CC // 2, _step, None)
        plsc.subcore_barrier()
        pltpu.sync_copy(acc_sh.at[pl.ds(s * _RPT, _RPT)],
                        acc_hbm.at[qg, pl.ds(s * _RPT, _RPT)])
        plsc.subcore_barrier()


def _msg_call(*args):
    return pl.kernel(
        _msg_body,
        out_type=jax.ShapeDtypeStruct((_NQ, _NP, _DQ), jnp.float32),
        mesh=_make_mesh(),
        compiler_params=pltpu.CompilerParams(use_tc_tiling_on_sc=False),
        scratch_types=[
            pltpu.VMEM((_CC, _KC), jnp.int32),
            pltpu.VMEM((_CC, _KC), jnp.int32),
            pltpu.VMEM((_KC, _DQ), jnp.float32),
            pltpu.VMEM((_KC, _DQ), jnp.float32),
            pltpu.VMEM_SHARED((_NP, _DQ), jnp.float32),
            pltpu.SemaphoreType.DMA,
            pltpu.SemaphoreType.DMA,
        ],
    )(*args)


def _fin_body(acc_ref, g_ref, deg_ref, b_ref,
              a1_refs, a2_refs, a3_refs, a4_refs,
              bb1, bb2, bb3, bb4, o1, o2, o3, o4, s_acc):
    i = pl.program_id(0)
    deg = deg_ref[0, :, 0:1] + deg_ref[1, :, 0:1] + 1.0
    dinv = lax.rsqrt(deg)

    @pl.when(i == 0)
    def _():
        s_acc[...] = jnp.zeros_like(s_acc)

    for q in range(_NQ):
        t = jnp.maximum(
            dinv * (acc_ref[q] + g_ref[q]) + b_ref[q:q + 1, :], 0.0)
        s_acc[q:q + 1, :] += jnp.sum(t, axis=0, keepdims=True)

    @pl.when(i == _NB - 1)
    def _():
        def head(a_refs, bb):
            r = bb[...]
            for q in range(_NQ):
                r = r + jnp.dot(s_acc[q:q + 1, :], a_refs[q][...],
                                preferred_element_type=jnp.float32)
            return jnp.tanh(r)

        o1[...] = head(a1_refs, bb1)
        o2[...] = head(a2_refs, bb2)
        o3[...] = head(a3_refs, bb3)
        o4[...] = head(a4_refs, bb4)


def _fin_wrap(acc_ref, g_ref, deg_ref, b_ref,
              a10, a11, a12, a13, a20, a21, a22, a23,
              a30, a31, a32, a33, a40, a41, a42, a43,
              bb1, bb2, bb3, bb4, o1, o2, o3, o4, s_acc):
    _fin_body(acc_ref, g_ref, deg_ref, b_ref,
              (a10, a11, a12, a13), (a20, a21, a22, a23),
              (a30, a31, a32, a33), (a40, a41, a42, a43),
              bb1, bb2, bb3, bb4, o1, o2, o3, o4, s_acc)


def _const_spec(shape):
    return pl.BlockSpec(shape, lambda i: tuple(0 for _ in shape))


_fin_call = pl.pallas_call(
    _fin_wrap,
    grid=(_NB,),
    in_specs=[
        pl.BlockSpec((_NQ, _BN, _DQ), lambda i: (0, i, 0)),
        pl.BlockSpec((_NQ, _BN, _DQ), lambda i: (0, i, 0)),
        pl.BlockSpec((_NC, _BN, 16), lambda i: (0, i, 0)),
        _const_spec((_NQ, _DQ)),
    ]
    + [_const_spec((_DQ, _D2))] * 8
    + [_const_spec((_DQ, _DIN))] * 8
    + [
        _const_spec((1, _D2)), _const_spec((1, _D2)),
        _const_spec((1, _DIN)), _const_spec((1, _DIN)),
    ],
    out_specs=[
        _const_spec((1, _D2)), _const_spec((1, _D2)),
        _const_spec((1, _DIN)), _const_spec((1, _DIN)),
    ],
    out_shape=[
        jax.ShapeDtypeStruct((1, _D2), jnp.float32),
        jax.ShapeDtypeStruct((1, _D2), jnp.float32),
        jax.ShapeDtypeStruct((1, _DIN), jnp.float32),
        jax.ShapeDtypeStruct((1, _DIN), jnp.float32),
    ],
    scratch_shapes=[pltpu.VMEM((_NQ, _DQ), jnp.float32)],
)


def kernel(x, edge_index, conv1_weight, conv1_bias,
           fc1_weight, fc1_bias, fc2_weight, fc2_bias,
           fc3_weight, fc3_bias, fc4_weight, fc4_bias):
    src = edge_index[0]
    dst = edge_index[1]
    src_a = src.reshape(_NW, _CA, _KA)
    src_c = src.reshape(_NS, _CC, _KC)
    dst_c = dst.reshape(_NS, _CC, _KC)
    ones_a = jnp.ones((_KA, 16), jnp.float32)
    zeros_a = jnp.zeros((_NP, 16), jnp.float32)
    zeros_c = jnp.zeros((_NP, _DQ), jnp.float32)

    deg2 = _deg_call(src_a, ones_a, zeros_a)
    g4 = _mm_call(x, conv1_weight, deg2)
    acc4 = _msg_call(g4, src_c, dst_c, zeros_c)

    b4 = conv1_bias.reshape(_NQ, _DQ)
    legs = []
    for w in (fc1_weight, fc2_weight, fc3_weight, fc4_weight):
        for q in range(_NQ):
            legs.append(w[:, q * _DQ:(q + 1) * _DQ].T)
    bb1 = fc1_bias.reshape(1, _D2)
    bb2 = fc2_bias.reshape(1, _D2)
    bb3 = fc3_bias.reshape(1, _DIN)
    bb4 = fc4_bias.reshape(1, _DIN)

    o1, o2, o3, o4 = _fin_call(acc4, g4, deg2, b4, *legs,
                               bb1, bb2, bb3, bb4)
    return (o1.reshape(_D2), o2.reshape(_D2),
            o3.reshape(_DIN), o4.reshape(_DIN))


# speedup vs baseline: 27.8073x; 1.3068x over previous
import jax
import jax.numpy as jnp
from jax import lax
from jax.experimental import pallas as pl
from jax.experimental.pallas import tpu as pltpu
from jax.experimental.pallas import tpu_sc as plsc

_N = 10000
_NP = 10240
_E = 320000
_DIN = 128
_D2 = 256
_DQ = 64
_NQ = 4
_NC = 2
_NS = 16
_NW = _NC * _NS
_RPT = _NP // _NS
_KA = 80
_CA = (_E // _NW) // _KA
_KC = 125
_CC = (_E // _NS) // _KC
_BN = 1000
_NB = _N // _BN

def _make_mesh():
    return plsc.VectorSubcoreMesh(core_axis_name="c", subcore_axis_name="s",
                                  num_cores=_NC, num_subcores=_NS)


def _deg_body(src_hbm, ones_hbm, zeros_hbm, deg_hbm, idx_v, ones_v, deg_sh):
    c = lax.axis_index("c")
    s = lax.axis_index("s")
    w = c * _NS + s
    pltpu.sync_copy(zeros_hbm.at[pl.ds(s * _RPT, _RPT), pl.ds(0, 16)],
                    deg_sh.at[pl.ds(s * _RPT, _RPT)])
    pltpu.sync_copy(ones_hbm, ones_v)
    pltpu.sync_copy(src_hbm.at[w], idx_v)
    plsc.subcore_barrier()

    def _chunk(j, carry):
        pltpu.sync_copy(ones_v, deg_sh.at[idx_v.at[j]], add=True)
        return carry

    lax.fori_loop(0, _CA, _chunk, None)
    plsc.subcore_barrier()
    pltpu.sync_copy(deg_sh.at[pl.ds(s * _RPT, _RPT)],
                    deg_hbm.at[c, pl.ds(s * _RPT, _RPT)])


def _deg_call(*args):
    return pl.kernel(
        _deg_body,
        out_type=jax.ShapeDtypeStruct((_NC, _NP, 16), jnp.float32),
        mesh=_make_mesh(),
        compiler_params=pltpu.CompilerParams(use_tc_tiling_on_sc=False),
        scratch_types=[
            pltpu.VMEM((_CA, _KA), jnp.int32),
            pltpu.VMEM((_KA, 16), jnp.float32),
            pltpu.VMEM_SHARED((_NP, 16), jnp.float32),
        ],
    )(*args)


def _mm_body(x_ref, w_ref, deg_ref, g4_ref):
    h = jnp.dot(x_ref[...], w_ref[...], preferred_element_type=jnp.float32)
    deg = deg_ref[0, :, 0:1] + deg_ref[1, :, 0:1] + 1.0
    g = lax.rsqrt(deg) * h
    for q in range(_NQ):
        g4_ref[q] = g[:, q * _DQ:(q + 1) * _DQ]


_mm_call = pl.pallas_call(
    _mm_body,
    grid=(_NB,),
    in_specs=[
        pl.BlockSpec((_BN, _DIN), lambda i: (i, 0)),
        pl.BlockSpec((_DIN, _D2), lambda i: (0, 0)),
        pl.BlockSpec((_NC, _BN, 16), lambda i: (0, i, 0)),
    ],
    out_specs=pl.BlockSpec((_NQ, _BN, _DQ), lambda i: (0, i, 0)),
    out_shape=jax.ShapeDtypeStruct((_NQ, _NP, _DQ), jnp.float32),
)


def _msg_body(g4_hbm, src_hbm, dst_hbm, zeros_hbm, acc_hbm,
              src_v, dst_v, r0, r1, r2, r3, acc_sh,
              g0, g1, g2, g3, s0, s1, s2, s3):
    c = lax.axis_index("c")
    s = lax.axis_index("s")
    rows = (r0, r1, r2, r3)
    gsem = (g0, g1, g2, g3)
    ssem = (s0, s1, s2, s3)
    pltpu.sync_copy(src_hbm.at[s], src_v)
    pltpu.sync_copy(dst_hbm.at[s], dst_v)

    for q in range(2):
        qg = c * 2 + q
        pltpu.sync_copy(zeros_hbm.at[pl.ds(s * _RPT, _RPT)],
                        acc_sh.at[pl.ds(s * _RPT, _RPT)])
        plsc.subcore_barrier()

        tbl = g4_hbm.at[qg]
        for b in range(3):
            pltpu.async_copy(tbl.at[src_v.at[b]], rows[b], gsem[b])

        def _step(t, carry):
            for b in range(4):
                j = 4 * t + b
                pltpu.make_async_copy(tbl.at[src_v.at[j]],
                                      rows[b], gsem[b]).wait()
                pltpu.async_copy(rows[b], acc_sh.at[dst_v.at[j]],
                                 ssem[b], add=True)
                jn = j + 3
                bn = (b + 3) % 4
                cond = jn < _CC if b else jnp.logical_and(t > 0, jn < _CC)

                @pl.when(cond)
                def _():
                    pltpu.make_async_copy(
                        rows[bn], acc_sh.at[dst_v.at[j - 1]],
                        ssem[bn]).wait()
                    pltpu.async_copy(tbl.at[src_v.at[jn]],
                                     rows[bn], gsem[bn])
                if b == 0:
                    @pl.when(t == 0)
                    def _():
                        pltpu.async_copy(tbl.at[src_v.at[3]],
                                         rows[3], gsem[3])
            return carry

        lax.fori_loop(0, _CC // 4, _step, None)
        for b in range(4):
            pltpu.make_async_copy(rows[b],
                                  acc_sh.at[dst_v.at[_CC - 4 + b]],
                                  ssem[b]).wait()
        plsc.subcore_barrier()
        pltpu.sync_copy(acc_sh.at[pl.ds(s * _RPT, _RPT)],
                        acc_hbm.at[qg, pl.ds(s * _RPT, _RPT)])
        plsc.subcore_barrier()


def _msg_call(*args):
    return pl.kernel(
        _msg_body,
        out_type=jax.ShapeDtypeStruct((_NQ, _NP, _DQ), jnp.float32),
        mesh=_make_mesh(),
        compiler_params=pltpu.CompilerParams(use_tc_tiling_on_sc=False),
        scratch_types=[
            pltpu.VMEM((_CC, _KC), jnp.int32),
            pltpu.VMEM((_CC, _KC), jnp.int32),
        ] + [pltpu.VMEM((_KC, _DQ), jnp.float32)] * 4 + [
            pltpu.VMEM_SHARED((_NP, _DQ), jnp.float32),
        ] + [pltpu.SemaphoreType.DMA] * 8,
    )(*args)


def _fin_body(acc_ref, g_ref, deg_ref, b_ref,
              a1_refs, a2_refs, a3_refs, a4_refs,
              bb1, bb2, bb3, bb4, o1, o2, o3, o4, s_acc):
    i = pl.program_id(0)
    deg = deg_ref[0, :, 0:1] + deg_ref[1, :, 0:1] + 1.0
    dinv = lax.rsqrt(deg)

    @pl.when(i == 0)
    def _():
        s_acc[...] = jnp.zeros_like(s_acc)

    for q in range(_NQ):
        t = jnp.maximum(
            dinv * (acc_ref[q] + g_ref[q]) + b_ref[q:q + 1, :], 0.0)
        s_acc[q:q + 1, :] += jnp.sum(t, axis=0, keepdims=True)

    @pl.when(i == _NB - 1)
    def _():
        def head(a_refs, bb):
            r = bb[...]
            for q in range(_NQ):
                r = r + jnp.dot(s_acc[q:q + 1, :], a_refs[q][...],
                                preferred_element_type=jnp.float32)
            return jnp.tanh(r)

        o1[...] = head(a1_refs, bb1)
        o2[...] = head(a2_refs, bb2)
        o3[...] = head(a3_refs, bb3)
        o4[...] = head(a4_refs, bb4)


def _fin_wrap(acc_ref, g_ref, deg_ref, b_ref,
              a10, a11, a12, a13, a20, a21, a22, a23,
              a30, a31, a32, a33, a40, a41, a42, a43,
              bb1, bb2, bb3, bb4, o1, o2, o3, o4, s_acc):
    _fin_body(acc_ref, g_ref, deg_ref, b_ref,
              (a10, a11, a12, a13), (a20, a21, a22, a23),
              (a30, a31, a32, a33), (a40, a41, a42, a43),
              bb1, bb2, bb3, bb4, o1, o2, o3, o4, s_acc)


def _const_spec(shape):
    return pl.BlockSpec(shape, lambda i: tuple(0 for _ in shape))


_fin_call = pl.pallas_call(
    _fin_wrap,
    grid=(_NB,),
    in_specs=[
        pl.BlockSpec((_NQ, _BN, _DQ), lambda i: (0, i, 0)),
        pl.BlockSpec((_NQ, _BN, _DQ), lambda i: (0, i, 0)),
        pl.BlockSpec((_NC, _BN, 16), lambda i: (0, i, 0)),
        _const_spec((_NQ, _DQ)),
    ]
    + [_const_spec((_DQ, _D2))] * 8
    + [_const_spec((_DQ, _DIN))] * 8
    + [
        _const_spec((1, _D2)), _const_spec((1, _D2)),
        _const_spec((1, _DIN)), _const_spec((1, _DIN)),
    ],
    out_specs=[
        _const_spec((1, _D2)), _const_spec((1, _D2)),
        _const_spec((1, _DIN)), _const_spec((1, _DIN)),
    ],
    out_shape=[
        jax.ShapeDtypeStruct((1, _D2), jnp.float32),
        jax.ShapeDtypeStruct((1, _D2), jnp.float32),
        jax.ShapeDtypeStruct((1, _DIN), jnp.float32),
        jax.ShapeDtypeStruct((1, _DIN), jnp.float32),
    ],
    scratch_shapes=[pltpu.VMEM((_NQ, _DQ), jnp.float32)],
)


def kernel(x, edge_index, conv1_weight, conv1_bias,
           fc1_weight, fc1_bias, fc2_weight, fc2_bias,
           fc3_weight, fc3_bias, fc4_weight, fc4_bias):
    src = edge_index[0]
    dst = edge_index[1]
    src_a = src.reshape(_NW, _CA, _KA)
    src_c = src.reshape(_NS, _CC, _KC)
    dst_c = dst.reshape(_NS, _CC, _KC)
    ones_a = jnp.ones((_KA, 16), jnp.float32)
    zeros_a = jnp.zeros((_NP, 16), jnp.float32)
    zeros_c = jnp.zeros((_NP, _DQ), jnp.float32)

    deg2 = _deg_call(src_a, ones_a, zeros_a)
    g4 = _mm_call(x, conv1_weight, deg2)
    acc4 = _msg_call(g4, src_c, dst_c, zeros_c)

    b4 = conv1_bias.reshape(_NQ, _DQ)
    legs = []
    for w in (fc1_weight, fc2_weight, fc3_weight, fc4_weight):
        for q in range(_NQ):
            legs.append(w[:, q * _DQ:(q + 1) * _DQ].T)
    bb1 = fc1_bias.reshape(1, _D2)
    bb2 = fc2_bias.reshape(1, _D2)
    bb3 = fc3_bias.reshape(1, _DIN)
    bb4 = fc4_bias.reshape(1, _DIN)

    o1, o2, o3, o4 = _fin_call(acc4, g4, deg2, b4, *legs,
                               bb1, bb2, bb3, bb4)
    return (o1.reshape(_D2), o2.reshape(_D2),
            o3.reshape(_DIN), o4.reshape(_DIN))


# speedup vs baseline: 39.9008x; 1.4349x over previous
import jax
import jax.numpy as jnp
from jax import lax
from jax.experimental import pallas as pl
from jax.experimental.pallas import tpu as pltpu
from jax.experimental.pallas import tpu_sc as plsc

_N = 10000
_NP = 10240
_E = 320000
_DIN = 128
_D2 = 256
_DH = 128
_NC = 2
_NS = 16
_NW = _NC * _NS
_RPT = _NP // _NS
_KA = 125
_CA = (_E // _NW) // _KA
_KC = 125
_CC = (_E // _NS) // _KC
_BN = 1000
_NB = _N // _BN


def _make_mesh():
    return plsc.VectorSubcoreMesh(core_axis_name="c", subcore_axis_name="s",
                                  num_cores=_NC, num_subcores=_NS)


def _deg_body(src_hbm, ones_hbm, zeros_hbm, deg_hbm, idx_v, ones_v, deg_sh):
    c = lax.axis_index("c")
    s = lax.axis_index("s")
    w = c * _NS + s
    pltpu.sync_copy(zeros_hbm.at[pl.ds(s * _RPT, _RPT)],
                    deg_sh.at[pl.ds(s * _RPT, _RPT)])
    pltpu.sync_copy(ones_hbm, ones_v)
    pltpu.sync_copy(src_hbm.at[w], idx_v)
    plsc.subcore_barrier()

    def _chunk(j, carry):
        pltpu.sync_copy(ones_v, deg_sh.at[idx_v.at[j]], add=True)
        return carry

    lax.fori_loop(0, _CA, _chunk, None)
    plsc.subcore_barrier()
    pltpu.sync_copy(deg_sh.at[pl.ds(s * _RPT, _RPT)],
                    deg_hbm.at[c, pl.ds(s * _RPT, _RPT)])


def _deg_call(*args):
    return pl.kernel(
        _deg_body,
        out_type=jax.ShapeDtypeStruct((_NC, _NP, 16), jnp.float32),
        mesh=_make_mesh(),
        compiler_params=pltpu.CompilerParams(use_tc_tiling_on_sc=False),
        scratch_types=[
            pltpu.VMEM((_CA, _KA), jnp.int32),
            pltpu.VMEM((_KA, 16), jnp.float32),
            pltpu.VMEM_SHARED((_NP, 16), jnp.float32),
        ],
    )(*args)


def _mm_body(x_ref, w_ref, deg_ref, g2_ref):
    h = jnp.dot(x_ref[...], w_ref[...], preferred_element_type=jnp.float32)
    deg = deg_ref[0, :, 0:1] + deg_ref[1, :, 0:1] + 1.0
    g = lax.rsqrt(deg) * h
    g2_ref[0] = g[:, :_DH].astype(jnp.bfloat16)
    g2_ref[1] = g[:, _DH:].astype(jnp.bfloat16)


_mm_call = pl.pallas_call(
    _mm_body,
    grid=(_NB,),
    in_specs=[
        pl.BlockSpec((_BN, _DIN), lambda i: (i, 0)),
        pl.BlockSpec((_DIN, _D2), lambda i: (0, 0)),
        pl.BlockSpec((_NC, _BN, 16), lambda i: (0, i, 0)),
    ],
    out_specs=pl.BlockSpec((_NC, _BN, _DH), lambda i: (0, i, 0)),
    out_shape=jax.ShapeDtypeStruct((_NC, _NP, _DH), jnp.bfloat16),
)


def _msg_body(g2_hbm, src_hbm, dst_hbm, zeros_hbm, acc_hbm,
              src_v, dst_v, r0, r1, r2, r3, acc_sh,
              g0, g1, g2, g3, s0, s1, s2, s3):
    c = lax.axis_index("c")
    s = lax.axis_index("s")
    rows = (r0, r1, r2, r3)
    gsem = (g0, g1, g2, g3)
    ssem = (s0, s1, s2, s3)
    pltpu.sync_copy(src_hbm.at[s], src_v)
    pltpu.sync_copy(dst_hbm.at[s], dst_v)
    pltpu.sync_copy(zeros_hbm.at[pl.ds(s * _RPT, _RPT)],
                    acc_sh.at[pl.ds(s * _RPT, _RPT)])
    plsc.subcore_barrier()

    tbl = g2_hbm.at[c]
    for b in range(3):
        pltpu.async_copy(tbl.at[src_v.at[b]], rows[b], gsem[b])

    def _step(t, carry):
        for b in range(4):
            j = 4 * t + b
            pltpu.make_async_copy(tbl.at[src_v.at[j]],
                                  rows[b], gsem[b]).wait()
            pltpu.async_copy(rows[b], acc_sh.at[dst_v.at[j]],
                             ssem[b], add=True)
            jn = j + 3
            bn = (b + 3) % 4
            cond = jn < _CC if b else jnp.logical_and(t > 0, jn < _CC)

            @pl.when(cond)
            def _():
                pltpu.make_async_copy(
                    rows[bn], acc_sh.at[dst_v.at[j - 1]],
                    ssem[bn]).wait()
                pltpu.async_copy(tbl.at[src_v.at[jn]],
                                 rows[bn], gsem[bn])
            if b == 0:
                @pl.when(t == 0)
                def _():
                    pltpu.async_copy(tbl.at[src_v.at[3]],
                                     rows[3], gsem[3])
        return carry

    lax.fori_loop(0, _CC // 4, _step, None)
    for b in range(4):
        pltpu.make_async_copy(rows[b],
                              acc_sh.at[dst_v.at[_CC - 4 + b]],
                              ssem[b]).wait()
    plsc.subcore_barrier()
    pltpu.sync_copy(acc_sh.at[pl.ds(s * _RPT, _RPT)],
                    acc_hbm.at[c, pl.ds(s * _RPT, _RPT)])


def _msg_call(*args):
    return pl.kernel(
        _msg_body,
        out_type=jax.ShapeDtypeStruct((_NC, _NP, _DH), jnp.bfloat16),
        mesh=_make_mesh(),
        compiler_params=pltpu.CompilerParams(use_tc_tiling_on_sc=False),
        scratch_types=[
            pltpu.VMEM((_CC, _KC), jnp.int32),
            pltpu.VMEM((_CC, _KC), jnp.int32),
        ] + [pltpu.VMEM((_KC, _DH), jnp.bfloat16)] * 4 + [
            pltpu.VMEM_SHARED((_NP, _DH), jnp.bfloat16),
        ] + [pltpu.SemaphoreType.DMA] * 8,
    )(*args)


def _fin_body(x_ref, w_ref, acc_ref, deg_ref, b_ref,
              a10, a11, a20, a21, a30, a31, a40, a41,
              bb1, bb2, bb3, bb4, o1, o2, o3, o4, s_acc):
    i = pl.program_id(0)
    h = jnp.dot(x_ref[...], w_ref[...], preferred_element_type=jnp.float32)
    deg = deg_ref[0, :, 0:1] + deg_ref[1, :, 0:1] + 1.0
    dinv = lax.rsqrt(deg)
    g = dinv * h

    @pl.when(i == 0)
    def _():
        s_acc[...] = jnp.zeros_like(s_acc)

    for q in range(2):
        t = jnp.maximum(
            dinv * acc_ref[q].astype(jnp.float32)
            + dinv * g[:, q * _DH:(q + 1) * _DH]
            + b_ref[q:q + 1, :], 0.0)
        s_acc[q:q + 1, :] += jnp.sum(t, axis=0, keepdims=True)

    @pl.when(i == _NB - 1)
    def _():
        def head(al, ar, bb):
            return jnp.tanh(
                jnp.dot(s_acc[0:1, :], al[...],
                        preferred_element_type=jnp.float32)
                + jnp.dot(s_acc[1:2, :], ar[...],
                          preferred_element_type=jnp.float32)
                + bb[...])

        o1[...] = head(a10, a11, bb1)
        o2[...] = head(a20, a21, bb2)
        o3[...] = head(a30, a31, bb3)
        o4[...] = head(a40, a41, bb4)


def _const_spec(shape):
    return pl.BlockSpec(shape, lambda i: tuple(0 for _ in shape))


_fin_call = pl.pallas_call(
    _fin_body,
    grid=(_NB,),
    in_specs=[
        pl.BlockSpec((_BN, _DIN), lambda i: (i, 0)),
        pl.BlockSpec((_DIN, _D2), lambda i: (0, 0)),
        pl.BlockSpec((_NC, _BN, _DH), lambda i: (0, i, 0)),
        pl.BlockSpec((_NC, _BN, 16), lambda i: (0, i, 0)),
        _const_spec((_NC, _DH)),
    ]
    + [_const_spec((_DH, _D2))] * 4
    + [_const_spec((_DH, _DIN))] * 4
    + [
        _const_spec((1, _D2)), _const_spec((1, _D2)),
        _const_spec((1, _DIN)), _const_spec((1, _DIN)),
    ],
    out_specs=[
        _const_spec((1, _D2)), _const_spec((1, _D2)),
        _const_spec((1, _DIN)), _const_spec((1, _DIN)),
    ],
    out_shape=[
        jax.ShapeDtypeStruct((1, _D2), jnp.float32),
        jax.ShapeDtypeStruct((1, _D2), jnp.float32),
        jax.ShapeDtypeStruct((1, _DIN), jnp.float32),
        jax.ShapeDtypeStruct((1, _DIN), jnp.float32),
    ],
    scratch_shapes=[pltpu.VMEM((_NC, _DH), jnp.float32)],
)


def kernel(x, edge_index, conv1_weight, conv1_bias,
           fc1_weight, fc1_bias, fc2_weight, fc2_bias,
           fc3_weight, fc3_bias, fc4_weight, fc4_bias):
    src = edge_index[0]
    dst = edge_index[1]
    src_a = src.reshape(_NW, _CA, _KA)
    src_c = src.reshape(_NS, _CC, _KC)
    dst_c = dst.reshape(_NS, _CC, _KC)
    ones_a = jnp.ones((_KA, 16), jnp.float32)
    zeros_a = jnp.zeros((_NP, 16), jnp.float32)
    zeros_c = jnp.zeros((_NP, _DH), jnp.bfloat16)

    deg2 = _deg_call(src_a, ones_a, zeros_a)
    g2 = _mm_call(x, conv1_weight, deg2)
    acc2 = _msg_call(g2, src_c, dst_c, zeros_c)

    b2 = conv1_bias.reshape(_NC, _DH)
    a10 = fc1_weight[:, :_DH].T
    a11 = fc1_weight[:, _DH:].T
    a20 = fc2_weight[:, :_DH].T
    a21 = fc2_weight[:, _DH:].T
    a30 = fc3_weight[:, :_DH].T
    a31 = fc3_weight[:, _DH:].T
    a40 = fc4_weight[:, :_DH].T
    a41 = fc4_weight[:, _DH:].T
    bb1 = fc1_bias.reshape(1, _D2)
    bb2 = fc2_bias.reshape(1, _D2)
    bb3 = fc3_bias.reshape(1, _DIN)
    bb4 = fc4_bias.reshape(1, _DIN)

    o1, o2, o3, o4 = _fin_call(x, conv1_weight, acc2, deg2, b2,
                               a10, a11, a20, a21, a30, a31, a40, a41,
                               bb1, bb2, bb3, bb4)
    return (o1.reshape(_D2), o2.reshape(_D2),
            o3.reshape(_DIN), o4.reshape(_DIN))
